# baseline (speedup 1.0000x reference)
import functools

import jax
import jax.numpy as jnp
from jax import lax
from jax.experimental import pallas as pl
from jax.experimental.pallas import tpu as pltpu
from jax.experimental.pallas import tpu_sc as plsc

N = 10000
E = 320000
D = 128
H = 4
DH = D // H
ALPHA = 0.2
EPS = 1e-15

NC = 2
NS = 16
NW = NC * NS
EPT = E // NW
ROW = 80
NCHUNK = EPT // ROW
NPAIR = (NCHUNK - 1) // 2
NPAD = 10240
RPS = NPAD // NS
G16 = ROW // 16


def _prep_body(x_ref, wall_ref, ab_ref, ht_ref, st_ref):
  x = x_ref[...]
  nrm = jnp.maximum(jnp.sqrt(jnp.sum(x * x, axis=1, keepdims=True)), EPS)
  r = jnp.clip(nrm, -1.0 + 1e-5, 1.0 - 1e-5)
  at = 0.5 * (jnp.log1p(r) - jnp.log1p(-r))
  xt = x / nrm * at
  h = jnp.dot(xt, wall_ref[...], preferred_element_type=jnp.float32)
  ht_ref[...] = h
  st_ref[...] = jnp.dot(h, ab_ref[...], preferred_element_type=jnp.float32)


def _post_body(hp_ref, rs_ref, rep_ref, out_ref, rs8_ref):
  acc = (hp_ref[0] + hp_ref[1])[:N]
  rsum4 = (rs_ref[0:H] + rs_ref[H:2 * H])[:, :N]
  rs8_ref[...] = jnp.concatenate(
      [rsum4, jnp.zeros((8 - H, N), jnp.float32)], axis=0)
  den = lax.dot_general(
      rsum4 + 1e-16, rep_ref[...], (((0,), (0,)), ((), ())),
      preferred_element_type=jnp.float32)
  sup = acc / den
  sup = jnp.where(sup > 0, sup, jnp.exp(jnp.minimum(sup, 0.0)) - 1.0)
  snrm = jnp.maximum(jnp.sqrt(jnp.sum(sup * sup, axis=1, keepdims=True)), EPS)
  ex = jnp.tanh(snrm) * sup / snrm
  enrm = jnp.maximum(jnp.sqrt(jnp.sum(ex * ex, axis=1, keepdims=True)), EPS)
  maxn = 1.0 - 4e-3
  out_ref[...] = jnp.where(enrm > maxn, ex / enrm * maxn, ex)


def _edge_body(ht_hbm, st_hbm, ei_hbm,
               ee_out, hp_out, rs_out,
               idx0, idx1, svs0, svs1, svd0, svd1, rows0, rows1, ebt0, ebt1,
               ebf0, ebf1, rixa0, rixa1, rixb0, rixb1, rixc0, rixc1,
               rixd0, rixd1, hp_sh, rs_sh, sem0, sem1):
  c = lax.axis_index("c")
  s = lax.axis_index("s")
  wid = c * NS + s
  iota = lax.iota(jnp.int32, 16)
  zeros16 = jnp.zeros((16,), jnp.float32)
  idxb = (idx0, idx1)
  svsb = (svs0, svs1)
  svdb = (svd0, svd1)
  rowsb = (rows0, rows1)
  ebtb = (ebt0, ebt1)
  ebfb = (ebf0, ebf1)
  rixb = ((rixa0, rixb0, rixc0, rixd0), (rixa1, rixb1, rixc1, rixd1))
  semb = (sem0, sem1)

  for r in range(ROW):
    for k in range(D // 16):
      rows0[r, pl.ds(k * 16, 16)] = zeros16
  for k in range(H * ROW // 16):
    ebf0[pl.ds(k * 16, 16)] = zeros16
  for k in range(RPS // ROW):
    pltpu.sync_copy(rows0, hp_sh.at[pl.ds(s * RPS + k * ROW, ROW)])
    pltpu.sync_copy(ebf0, rs_sh.at[pl.ds(s * H * RPS + k * H * ROW, H * ROW)])

  plsc.subcore_barrier()

  def load_chunk(ch, b):
    pltpu.sync_copy(ei_hbm.at[wid * NCHUNK + ch], idxb[b])
    cps = (pltpu.async_copy(ht_hbm.at[idxb[b].at[1]], rowsb[b], semb[b]),
           pltpu.async_copy(st_hbm.at[idxb[b].at[0]], svsb[b], semb[b]),
           pltpu.async_copy(st_hbm.at[idxb[b].at[1]], svdb[b], semb[b]))
    return cps

  def run_chunk(ch, b, cps):
    for cp in cps:
      cp.wait()

    def g_body(g, _):
      off = g * 16
      e16 = iota + off
      src16 = idxb[b][0, pl.ds(off, 16)]
      for hh in range(H):
        ssrc = plsc.load_gather(svsb[b], [e16, jnp.full((16,), hh, jnp.int32)])
        sdst = plsc.load_gather(svdb[b],
                                [e16, jnp.full((16,), H + hh, jnp.int32)])
        lg = ssrc + sdst
        ee = jnp.exp(-jnp.maximum(lg, ALPHA * lg))
        plsc.store_scatter(ebtb[b], [e16, jnp.full((16,), hh, jnp.int32)], ee)
        ebfb[b][pl.ds(hh * ROW + off, 16)] = ee
        rixb[b][hh][pl.ds(off, 16)] = src16 + hh * NPAD
      for j in range(16):
        ev = jnp.full((16,), off + j, jnp.int32)
        for hh in range(H):
          sc = plsc.load_gather(ebtb[b], [ev, jnp.full((16,), hh, jnp.int32)])
          for t in range(DH // 16):
            cv = iota + (hh * DH + t * 16)
            v = plsc.load_gather(rowsb[b], [ev, cv])
            plsc.store_scatter(rowsb[b], [ev, cv], v * sc)
      return 0

    lax.fori_loop(0, G16, g_body, 0)

    ebase = wid * EPT + ch * ROW
    pltpu.sync_copy(rowsb[b], hp_sh.at[idxb[b].at[0]], add=True)
    for hh in range(H):
      pltpu.sync_copy(ebfb[b].at[pl.ds(hh * ROW, ROW)],
                      rs_sh.at[rixb[b][hh]], add=True)
    pltpu.sync_copy(ebtb[b], ee_out.at[pl.ds(ebase, ROW)])

  def pair_body(i, _):
    ch = i * 2
    cps0 = load_chunk(ch, 0)
    cps1 = load_chunk(ch + 1, 1)
    run_chunk(ch, 0, cps0)
    run_chunk(ch + 1, 1, cps1)
    return 0

  lax.fori_loop(0, NPAIR, pair_body, 0)
  last = NPAIR * 2
  run_chunk(last, 0, load_chunk(last, 0))

  plsc.subcore_barrier()
  pltpu.sync_copy(hp_sh.at[pl.ds(s * RPS, RPS)],
                  hp_out.at[c, pl.ds(s * RPS, RPS)])
  pltpu.sync_copy(rs_sh.at[pl.ds(s * H * RPS, H * RPS)],
                  rs_out.at[c, 0, pl.ds(s * H * RPS, H * RPS)])


@functools.cache
def _edge_kernel():
  return functools.partial(
      pl.kernel,
      out_type=(jax.ShapeDtypeStruct((E, H), jnp.float32),
                jax.ShapeDtypeStruct((NC, NPAD, D), jnp.float32),
                jax.ShapeDtypeStruct((NC, 1, H * NPAD), jnp.float32)),
      mesh=plsc.VectorSubcoreMesh(core_axis_name="c", subcore_axis_name="s",
                                  num_cores=NC, num_subcores=NS),
      compiler_params=pltpu.CompilerParams(use_tc_tiling_on_sc=False,
                                           needs_layout_passes=False),
      scratch_types=[
          pltpu.VMEM((2, ROW), jnp.int32),
          pltpu.VMEM((2, ROW), jnp.int32),
          pltpu.VMEM((ROW, 2 * H), jnp.float32),
          pltpu.VMEM((ROW, 2 * H), jnp.float32),
          pltpu.VMEM((ROW, 2 * H), jnp.float32),
          pltpu.VMEM((ROW, 2 * H), jnp.float32),
          pltpu.VMEM((ROW, D), jnp.float32),
          pltpu.VMEM((ROW, D), jnp.float32),
          pltpu.VMEM((ROW, H), jnp.float32),
          pltpu.VMEM((ROW, H), jnp.float32),
          pltpu.VMEM((H * ROW,), jnp.float32),
          pltpu.VMEM((H * ROW,), jnp.float32),
          pltpu.VMEM((ROW,), jnp.int32),
          pltpu.VMEM((ROW,), jnp.int32),
          pltpu.VMEM((ROW,), jnp.int32),
          pltpu.VMEM((ROW,), jnp.int32),
          pltpu.VMEM((ROW,), jnp.int32),
          pltpu.VMEM((ROW,), jnp.int32),
          pltpu.VMEM((ROW,), jnp.int32),
          pltpu.VMEM((ROW,), jnp.int32),
          pltpu.VMEM_SHARED((NPAD, D), jnp.float32),
          pltpu.VMEM_SHARED((H * NPAD,), jnp.float32),
          pltpu.SemaphoreType.DMA,
          pltpu.SemaphoreType.DMA,
      ],
  )(_edge_body)


def kernel(x, edge_index, W, a):
  wall = jnp.transpose(W, (1, 0, 2)).reshape(D, D)
  ab = jnp.zeros((D, 2 * H), jnp.float32)
  for hh in range(H):
    ab = ab.at[hh * DH:(hh + 1) * DH, hh].set(a[hh, :DH])
    ab = ab.at[hh * DH:(hh + 1) * DH, H + hh].set(a[hh, DH:])
  ei3 = jnp.transpose(edge_index.reshape(2, E // ROW, ROW), (1, 0, 2))

  ht, st = pl.pallas_call(
      _prep_body,
      out_shape=(jax.ShapeDtypeStruct((N, D), jnp.float32),
                 jax.ShapeDtypeStruct((N, 2 * H), jnp.float32)),
  )(x, wall, ab)

  ee2, hp, rs = _edge_kernel()(ht, st, ei3)

  rep = jnp.zeros((H, D), jnp.float32)
  for hh in range(H):
    rep = rep.at[hh, hh * DH:(hh + 1) * DH].set(1.0)

  out, rs8 = pl.pallas_call(
      _post_body,
      out_shape=(jax.ShapeDtypeStruct((N, D), jnp.float32),
                 jax.ShapeDtypeStruct((8, N), jnp.float32)),
  )(hp, rs.reshape(NC * H, NPAD), rep)

  return out, ee2.T, rs8[:H]


# speedup vs baseline: 1.0225x; 1.0225x over previous
import functools

import jax
import jax.numpy as jnp
from jax import lax
from jax.experimental import pallas as pl
from jax.experimental.pallas import tpu as pltpu
from jax.experimental.pallas import tpu_sc as plsc

N = 10000
E = 320000
D = 128
H = 4
DH = D // H
ALPHA = 0.2
EPS = 1e-15

NC = 2
NS = 16
NW = NC * NS
EPT = E // NW
ROW = 80
NCHUNK = EPT // ROW
NPAIR = (NCHUNK - 1) // 2
NPAD = 10240
RPS = NPAD // NS
G16 = ROW // 16


def _prep_body(x_ref, wall_ref, ab_ref, ht_ref, st_ref):
  x = x_ref[...]
  nrm = jnp.maximum(jnp.sqrt(jnp.sum(x * x, axis=1, keepdims=True)), EPS)
  r = jnp.clip(nrm, -1.0 + 1e-5, 1.0 - 1e-5)
  at = 0.5 * (jnp.log1p(r) - jnp.log1p(-r))
  xt = x / nrm * at
  h = jnp.dot(xt, wall_ref[...], preferred_element_type=jnp.float32)
  ht_ref[...] = h
  st_ref[...] = jnp.dot(h, ab_ref[...], preferred_element_type=jnp.float32)


def _post_body(hp_ref, rs_ref, rep_ref, out_ref, rs8_ref):
  acc = (hp_ref[0] + hp_ref[1])[:N]
  rsum4 = (rs_ref[0:H] + rs_ref[H:2 * H])[:, :N]
  rs8_ref[...] = jnp.concatenate(
      [rsum4, jnp.zeros((8 - H, N), jnp.float32)], axis=0)
  den = lax.dot_general(
      rsum4 + 1e-16, rep_ref[...], (((0,), (0,)), ((), ())),
      preferred_element_type=jnp.float32)
  sup = acc / den
  sup = jnp.where(sup > 0, sup, jnp.exp(jnp.minimum(sup, 0.0)) - 1.0)
  snrm = jnp.maximum(jnp.sqrt(jnp.sum(sup * sup, axis=1, keepdims=True)), EPS)
  ex = jnp.tanh(snrm) * sup / snrm
  enrm = jnp.maximum(jnp.sqrt(jnp.sum(ex * ex, axis=1, keepdims=True)), EPS)
  maxn = 1.0 - 4e-3
  out_ref[...] = jnp.where(enrm > maxn, ex / enrm * maxn, ex)


def _edge_body(ht_hbm, st_hbm, ei_hbm,
               ee_out, hp_out, rs_out,
               idx0, idx1, svs0, svs1, svd0, svd1, rows0, rows1, ebt0, ebt1,
               ebf0, ebf1, rix0, rix1, hp_sh, rs_sh, sem0, sem1):
  c = lax.axis_index("c")
  s = lax.axis_index("s")
  wid = c * NS + s
  iota = lax.iota(jnp.int32, 16)
  zeros16 = jnp.zeros((16,), jnp.float32)
  idxb = (idx0, idx1)
  svsb = (svs0, svs1)
  svdb = (svd0, svd1)
  rowsb = (rows0, rows1)
  ebtb = (ebt0, ebt1)
  ebfb = (ebf0, ebf1)
  rixb = (rix0, rix1)
  semb = (sem0, sem1)

  for r in range(ROW):
    for k in range(D // 16):
      rows0[r, pl.ds(k * 16, 16)] = zeros16
  for k in range(H * ROW // 16):
    ebf0[pl.ds(k * 16, 16)] = zeros16
  for k in range(RPS // ROW):
    pltpu.sync_copy(rows0, hp_sh.at[pl.ds(s * RPS + k * ROW, ROW)])
    pltpu.sync_copy(ebf0, rs_sh.at[pl.ds(s * H * RPS + k * H * ROW, H * ROW)])

  plsc.subcore_barrier()

  def load_chunk(ch, b):
    pltpu.sync_copy(ei_hbm.at[wid * NCHUNK + ch], idxb[b])
    cps = (pltpu.async_copy(ht_hbm.at[idxb[b].at[1]], rowsb[b], semb[b]),
           pltpu.async_copy(st_hbm.at[idxb[b].at[0]], svsb[b], semb[b]),
           pltpu.async_copy(st_hbm.at[idxb[b].at[1]], svdb[b], semb[b]))
    return cps

  def run_chunk(ch, b, cps):
    for cp in cps:
      cp.wait()

    def g_body(g, _):
      off = g * 16
      e16 = iota + off
      src16 = idxb[b][0, pl.ds(off, 16)]
      for hh in range(H):
        ssrc = plsc.load_gather(svsb[b], [e16, jnp.full((16,), hh, jnp.int32)])
        sdst = plsc.load_gather(svdb[b],
                                [e16, jnp.full((16,), H + hh, jnp.int32)])
        lg = ssrc + sdst
        ee = jnp.exp(-jnp.maximum(lg, ALPHA * lg))
        plsc.store_scatter(ebtb[b], [e16, jnp.full((16,), hh, jnp.int32)], ee)
        ebfb[b][pl.ds(hh * ROW + off, 16)] = ee
        rixb[b][pl.ds(hh * ROW + off, 16)] = src16 + hh * NPAD
      for j in range(16):
        ev = jnp.full((16,), off + j, jnp.int32)
        for hh in range(H):
          sc = plsc.load_gather(ebtb[b], [ev, jnp.full((16,), hh, jnp.int32)])
          for t in range(DH // 16):
            cv = iota + (hh * DH + t * 16)
            v = plsc.load_gather(rowsb[b], [ev, cv])
            plsc.store_scatter(rowsb[b], [ev, cv], v * sc)
      return 0

    lax.fori_loop(0, G16, g_body, 0)

    ebase = wid * EPT + ch * ROW
    pltpu.sync_copy(rowsb[b], hp_sh.at[idxb[b].at[0]], add=True)
    pltpu.sync_copy(ebfb[b], rs_sh.at[rixb[b]], add=True)
    pltpu.sync_copy(ebtb[b], ee_out.at[pl.ds(ebase, ROW)])

  def pair_body(i, _):
    ch = i * 2
    cps0 = load_chunk(ch, 0)
    cps1 = load_chunk(ch + 1, 1)
    run_chunk(ch, 0, cps0)
    run_chunk(ch + 1, 1, cps1)
    return 0

  lax.fori_loop(0, NPAIR, pair_body, 0)
  last = NPAIR * 2
  run_chunk(last, 0, load_chunk(last, 0))

  plsc.subcore_barrier()
  pltpu.sync_copy(hp_sh.at[pl.ds(s * RPS, RPS)],
                  hp_out.at[c, pl.ds(s * RPS, RPS)])
  pltpu.sync_copy(rs_sh.at[pl.ds(s * H * RPS, H * RPS)],
                  rs_out.at[c, 0, pl.ds(s * H * RPS, H * RPS)])


@functools.cache
def _edge_kernel():
  return functools.partial(
      pl.kernel,
      out_type=(jax.ShapeDtypeStruct((E, H), jnp.float32),
                jax.ShapeDtypeStruct((NC, NPAD, D), jnp.float32),
                jax.ShapeDtypeStruct((NC, 1, H * NPAD), jnp.float32)),
      mesh=plsc.VectorSubcoreMesh(core_axis_name="c", subcore_axis_name="s",
                                  num_cores=NC, num_subcores=NS),
      compiler_params=pltpu.CompilerParams(use_tc_tiling_on_sc=False,
                                           needs_layout_passes=False),
      scratch_types=[
          pltpu.VMEM((2, ROW), jnp.int32),
          pltpu.VMEM((2, ROW), jnp.int32),
          pltpu.VMEM((ROW, 2 * H), jnp.float32),
          pltpu.VMEM((ROW, 2 * H), jnp.float32),
          pltpu.VMEM((ROW, 2 * H), jnp.float32),
          pltpu.VMEM((ROW, 2 * H), jnp.float32),
          pltpu.VMEM((ROW, D), jnp.float32),
          pltpu.VMEM((ROW, D), jnp.float32),
          pltpu.VMEM((ROW, H), jnp.float32),
          pltpu.VMEM((ROW, H), jnp.float32),
          pltpu.VMEM((H * ROW,), jnp.float32),
          pltpu.VMEM((H * ROW,), jnp.float32),
          pltpu.VMEM((H * ROW,), jnp.int32),
          pltpu.VMEM((H * ROW,), jnp.int32),
          pltpu.VMEM_SHARED((NPAD, D), jnp.float32),
          pltpu.VMEM_SHARED((H * NPAD,), jnp.float32),
          pltpu.SemaphoreType.DMA,
          pltpu.SemaphoreType.DMA,
      ],
  )(_edge_body)


def kernel(x, edge_index, W, a):
  wall = jnp.transpose(W, (1, 0, 2)).reshape(D, D)
  ab = jnp.zeros((D, 2 * H), jnp.float32)
  for hh in range(H):
    ab = ab.at[hh * DH:(hh + 1) * DH, hh].set(a[hh, :DH])
    ab = ab.at[hh * DH:(hh + 1) * DH, H + hh].set(a[hh, DH:])
  ei3 = jnp.transpose(edge_index.reshape(2, E // ROW, ROW), (1, 0, 2))

  ht, st = pl.pallas_call(
      _prep_body,
      out_shape=(jax.ShapeDtypeStruct((N, D), jnp.float32),
                 jax.ShapeDtypeStruct((N, 2 * H), jnp.float32)),
  )(x, wall, ab)

  ee2, hp, rs = _edge_kernel()(ht, st, ei3)

  rep = jnp.zeros((H, D), jnp.float32)
  for hh in range(H):
    rep = rep.at[hh, hh * DH:(hh + 1) * DH].set(1.0)

  out, rs8 = pl.pallas_call(
      _post_body,
      out_shape=(jax.ShapeDtypeStruct((N, D), jnp.float32),
                 jax.ShapeDtypeStruct((8, N), jnp.float32)),
  )(hp, rs.reshape(NC * H, NPAD), rep)

  return out, ee2.T, rs8[:H]


# speedup vs baseline: 1.0570x; 1.0337x over previous
import functools

import jax
import jax.numpy as jnp
from jax import lax
from jax.experimental import pallas as pl
from jax.experimental.pallas import tpu as pltpu
from jax.experimental.pallas import tpu_sc as plsc

N = 10000
E = 320000
D = 128
H = 4
DH = D // H
ALPHA = 0.2
EPS = 1e-15

NC = 2
NS = 16
NW = NC * NS
EPT = E // NW
ROW = 80
NCHUNK = EPT // ROW
NPAIR = (NCHUNK - 1) // 2
NPAD = 10240
RPS = NPAD // NS
G16 = ROW // 16


def _prep_body(x_ref, wall_ref, ab_ref, ht_ref, st_ref):
  x = x_ref[...]
  nrm = jnp.maximum(jnp.sqrt(jnp.sum(x * x, axis=1, keepdims=True)), EPS)
  r = jnp.clip(nrm, -1.0 + 1e-5, 1.0 - 1e-5)
  at = 0.5 * (jnp.log1p(r) - jnp.log1p(-r))
  xt = x / nrm * at
  h = jnp.dot(xt, wall_ref[...], preferred_element_type=jnp.float32)
  ht_ref[...] = h
  st_ref[...] = jnp.dot(h, ab_ref[...], preferred_element_type=jnp.float32)


def _post_body(hp_ref, rs_ref, rep_ref, out_ref, rs8_ref):
  acc = (hp_ref[0] + hp_ref[1])[:N]
  rsum4 = (rs_ref[0:H] + rs_ref[H:2 * H])[:, :N]
  rs8_ref[...] = jnp.concatenate(
      [rsum4, jnp.zeros((8 - H, N), jnp.float32)], axis=0)
  den = lax.dot_general(
      rsum4 + 1e-16, rep_ref[...], (((0,), (0,)), ((), ())),
      preferred_element_type=jnp.float32)
  sup = acc / den
  sup = jnp.where(sup > 0, sup, jnp.exp(jnp.minimum(sup, 0.0)) - 1.0)
  snrm = jnp.maximum(jnp.sqrt(jnp.sum(sup * sup, axis=1, keepdims=True)), EPS)
  ex = jnp.tanh(snrm) * sup / snrm
  enrm = jnp.maximum(jnp.sqrt(jnp.sum(ex * ex, axis=1, keepdims=True)), EPS)
  maxn = 1.0 - 4e-3
  out_ref[...] = jnp.where(enrm > maxn, ex / enrm * maxn, ex)


def _edge_body(ht_hbm, st_hbm, ei_hbm,
               ee_out, hp_out, rs_out,
               idx0, idx1, svs0, svs1, svd0, svd1, rows0, rows1, ebt0, ebt1,
               ebf0, ebf1, rix0, rix1, hp_sh, rs_sh, sem0, sem1):
  c = lax.axis_index("c")
  s = lax.axis_index("s")
  wid = c * NS + s
  iota = lax.iota(jnp.int32, 16)
  zeros16 = jnp.zeros((16,), jnp.float32)
  idxb = (idx0, idx1)
  svsb = (svs0, svs1)
  svdb = (svd0, svd1)
  rowsb = (rows0, rows1)
  ebtb = (ebt0, ebt1)
  ebfb = (ebf0, ebf1)
  rixb = (rix0, rix1)
  semb = (sem0, sem1)

  for r in range(ROW):
    for k in range(D // 16):
      rows0[r, pl.ds(k * 16, 16)] = zeros16
  for k in range(H * ROW // 16):
    ebf0[pl.ds(k * 16, 16)] = zeros16
  for k in range(RPS // ROW):
    pltpu.sync_copy(rows0, hp_sh.at[pl.ds(s * RPS + k * ROW, ROW)])
    pltpu.sync_copy(ebf0, rs_sh.at[pl.ds(s * H * RPS + k * H * ROW, H * ROW)])

  plsc.subcore_barrier()

  def load_chunk(ch, b):
    pltpu.sync_copy(ei_hbm.at[wid * NCHUNK + ch], idxb[b])
    cps = (pltpu.async_copy(ht_hbm.at[idxb[b].at[1]], rowsb[b], semb[b]),
           pltpu.async_copy(st_hbm.at[idxb[b].at[0]], svsb[b], semb[b]),
           pltpu.async_copy(st_hbm.at[idxb[b].at[1]], svdb[b], semb[b]))
    return cps

  def run_chunk(ch, b, cps):
    for cp in cps:
      cp.wait()

    def g_body(g, _):
      off = g * 16
      e16 = iota + off
      src16 = idxb[b][0, pl.ds(off, 16)]
      ee_h = []
      for hh in range(H):
        ssrc = plsc.load_gather(svsb[b], [e16, jnp.full((16,), hh, jnp.int32)])
        sdst = plsc.load_gather(svdb[b],
                                [e16, jnp.full((16,), H + hh, jnp.int32)])
        lg = ssrc + sdst
        ee = jnp.exp(-jnp.maximum(lg, ALPHA * lg))
        plsc.store_scatter(ebtb[b], [e16, jnp.full((16,), hh, jnp.int32)], ee)
        ebfb[b][pl.ds(hh * ROW + off, 16)] = ee
        rixb[b][pl.ds(hh * ROW + off, 16)] = src16 + hh * NPAD
        ee_h.append(ee)
      for j in range(16):
        ev = jnp.full((16,), off + j, jnp.int32)
        jv = jnp.full((16,), j, jnp.int32)
        for hh in range(H):
          sc = ee_h[hh].at[jv].get(mode="promise_in_bounds")
          for t in range(DH // 16):
            cv = iota + (hh * DH + t * 16)
            v = plsc.load_gather(rowsb[b], [ev, cv])
            plsc.store_scatter(rowsb[b], [ev, cv], v * sc)
      return 0

    lax.fori_loop(0, G16, g_body, 0)

    ebase = wid * EPT + ch * ROW
    pltpu.sync_copy(rowsb[b], hp_sh.at[idxb[b].at[0]], add=True)
    pltpu.sync_copy(ebfb[b], rs_sh.at[rixb[b]], add=True)
    pltpu.sync_copy(ebtb[b], ee_out.at[pl.ds(ebase, ROW)])

  def pair_body(i, _):
    ch = i * 2
    cps0 = load_chunk(ch, 0)
    cps1 = load_chunk(ch + 1, 1)
    run_chunk(ch, 0, cps0)
    run_chunk(ch + 1, 1, cps1)
    return 0

  lax.fori_loop(0, NPAIR, pair_body, 0)
  last = NPAIR * 2
  run_chunk(last, 0, load_chunk(last, 0))

  plsc.subcore_barrier()
  pltpu.sync_copy(hp_sh.at[pl.ds(s * RPS, RPS)],
                  hp_out.at[c, pl.ds(s * RPS, RPS)])
  pltpu.sync_copy(rs_sh.at[pl.ds(s * H * RPS, H * RPS)],
                  rs_out.at[c, 0, pl.ds(s * H * RPS, H * RPS)])


@functools.cache
def _edge_kernel():
  return functools.partial(
      pl.kernel,
      out_type=(jax.ShapeDtypeStruct((E, H), jnp.float32),
                jax.ShapeDtypeStruct((NC, NPAD, D), jnp.float32),
                jax.ShapeDtypeStruct((NC, 1, H * NPAD), jnp.float32)),
      mesh=plsc.VectorSubcoreMesh(core_axis_name="c", subcore_axis_name="s",
                                  num_cores=NC, num_subcores=NS),
      compiler_params=pltpu.CompilerParams(use_tc_tiling_on_sc=False,
                                           needs_layout_passes=False),
      scratch_types=[
          pltpu.VMEM((2, ROW), jnp.int32),
          pltpu.VMEM((2, ROW), jnp.int32),
          pltpu.VMEM((ROW, 2 * H), jnp.float32),
          pltpu.VMEM((ROW, 2 * H), jnp.float32),
          pltpu.VMEM((ROW, 2 * H), jnp.float32),
          pltpu.VMEM((ROW, 2 * H), jnp.float32),
          pltpu.VMEM((ROW, D), jnp.float32),
          pltpu.VMEM((ROW, D), jnp.float32),
          pltpu.VMEM((ROW, H), jnp.float32),
          pltpu.VMEM((ROW, H), jnp.float32),
          pltpu.VMEM((H * ROW,), jnp.float32),
          pltpu.VMEM((H * ROW,), jnp.float32),
          pltpu.VMEM((H * ROW,), jnp.int32),
          pltpu.VMEM((H * ROW,), jnp.int32),
          pltpu.VMEM_SHARED((NPAD, D), jnp.float32),
          pltpu.VMEM_SHARED((H * NPAD,), jnp.float32),
          pltpu.SemaphoreType.DMA,
          pltpu.SemaphoreType.DMA,
      ],
  )(_edge_body)


def kernel(x, edge_index, W, a):
  wall = jnp.transpose(W, (1, 0, 2)).reshape(D, D)
  ab = jnp.zeros((D, 2 * H), jnp.float32)
  for hh in range(H):
    ab = ab.at[hh * DH:(hh + 1) * DH, hh].set(a[hh, :DH])
    ab = ab.at[hh * DH:(hh + 1) * DH, H + hh].set(a[hh, DH:])
  ei3 = jnp.transpose(edge_index.reshape(2, E // ROW, ROW), (1, 0, 2))

  ht, st = pl.pallas_call(
      _prep_body,
      out_shape=(jax.ShapeDtypeStruct((N, D), jnp.float32),
                 jax.ShapeDtypeStruct((N, 2 * H), jnp.float32)),
  )(x, wall, ab)

  ee2, hp, rs = _edge_kernel()(ht, st, ei3)

  rep = jnp.zeros((H, D), jnp.float32)
  for hh in range(H):
    rep = rep.at[hh, hh * DH:(hh + 1) * DH].set(1.0)

  out, rs8 = pl.pallas_call(
      _post_body,
      out_shape=(jax.ShapeDtypeStruct((N, D), jnp.float32),
                 jax.ShapeDtypeStruct((8, N), jnp.float32)),
  )(hp, rs.reshape(NC * H, NPAD), rep)

  return out, ee2.T, rs8[:H]
